# baseline (speedup 1.0000x reference)
import functools

import jax
import jax.numpy as jnp
from jax import lax
from jax.experimental import pallas as pl
from jax.experimental.pallas import tpu as pltpu
from jax.experimental.pallas import tpu_sc as plsc

B, L, C = 4096, 20, 16
D = 32
VOC = 1000000
CHAR_VOC = 1000
CT_PITCH = 1024
WTROWS = VOC // 4

_info = plsc.get_sparse_core_info()
NC, NS, LANES = _info.num_cores, _info.num_subcores, _info.num_lanes
NW = NC * NS
RPW = B // NW
NG = RPW // LANES

_mesh = plsc.VectorSubcoreMesh(core_axis_name="c", subcore_axis_name="s")


@functools.partial(
    pl.kernel,
    out_type=jax.ShapeDtypeStruct((2 * D, B), jnp.float32),
    mesh=_mesh,
    compiler_params=pltpu.CompilerParams(use_tc_tiling_on_sc=True, needs_layout_passes=False),
    scratch_types=[
        pltpu.VMEM((LANES * CT_PITCH,), jnp.int32),
        pltpu.VMEM((L, RPW), jnp.int32),
        pltpu.VMEM((L, RPW), jnp.int32),
        pltpu.VMEM((L, C, RPW), jnp.int32),
        pltpu.VMEM((LANES, 4 * D), jnp.float32),
        pltpu.VMEM((LANES, 4 * D), jnp.float32),
        pltpu.VMEM((2 * D, RPW), jnp.float32),
        pltpu.SemaphoreType.DMA,
        pltpu.SemaphoreType.DMA,
    ],
)
def _emb_kernel(xt_hbm, xdiv_hbm, xct_hbm, wt4_hbm, ctp_hbm, out_hbm,
                ct_v, wi_v, wi4_v, ci_v, wr_a, wr_b, out_v, sem0, sem1):
    wid = lax.axis_index("s") * NC + lax.axis_index("c")
    base = wid * RPW
    pltpu.sync_copy(ctp_hbm, ct_v)
    pltpu.sync_copy(xt_hbm.at[:, pl.ds(base, RPW)], wi_v)
    pltpu.sync_copy(xdiv_hbm.at[:, pl.ds(base, RPW)], wi4_v)
    pltpu.sync_copy(xct_hbm.at[:, :, pl.ds(base, RPW)], ci_v)

    wscale = jnp.float32(1.0 / L)
    cscale = jnp.float32(1.0 / (L * C))
    zero = jnp.zeros((LANES,), jnp.float32)
    sh16 = jnp.int32(16)
    iota16 = lax.iota(jnp.int32, LANES)

    def fire(g, j, bufref, sem):
        pltpu.async_copy(
            wt4_hbm.at[wi4_v.at[j, pl.ds(g * LANES, LANES)]], bufref, sem)

    fire(0, 0, wr_a, sem0)
    fire(0, 1, wr_b, sem1)

    for g in range(NG):
        def cbody(k, acc):
            idxv = ci_v[k // C, k % C, pl.ds(g * LANES, LANES)]
            out = []
            for d in range(LANES):
                u = plsc.load_gather(ct_v, [idxv + (d * CT_PITCH)])
                lo = lax.bitcast_convert_type(u << sh16, jnp.float32)
                hi = lax.bitcast_convert_type(u, jnp.float32)
                out.append(acc[2 * d] + lo)
                out.append(acc[2 * d + 1] + hi)
            return tuple(out)

        acc = lax.fori_loop(0, L * C, cbody, (zero,) * (2 * LANES))
        for d in range(LANES):
            out_v[D + d, pl.ds(g * LANES, LANES)] = acc[2 * d] * cscale
            out_v[D + LANES + d, pl.ds(g * LANES, LANES)] = (
                acc[2 * d + 1] * cscale)

        def wbody(i, acc):
            out = acc
            for half, (bufref, sem) in enumerate(((wr_a, sem0), (wr_b, sem1))):
                j = i * 2 + half
                pltpu.make_async_copy(wt4_hbm.at[pl.ds(0, LANES)],
                                      bufref, sem).wait()
                wv = wi_v[j, pl.ds(g * LANES, LANES)]
                pos = (wv & 3) * D
                new = []
                for d in range(2 * LANES):
                    u = plsc.load_gather(bufref, [iota16, pos + d])
                    new.append(out[d] + u)
                out = tuple(new)
                nj = j + 2

                @pl.when(nj < L)
                def _():
                    fire(g, nj, bufref, sem)

                if g + 1 < NG:
                    @pl.when(nj >= L)
                    def _():
                        fire(g + 1, nj - L, bufref, sem)
            return out

        acc = lax.fori_loop(0, L // 2, wbody, (zero,) * (2 * LANES))
        for d in range(2 * LANES):
            out_v[d, pl.ds(g * LANES, LANES)] = acc[d] * wscale

    pltpu.sync_copy(out_v, out_hbm.at[:, pl.ds(base, RPW)])


def kernel(x, word_pos, x_char, x_mask, word_table, char_table):
    xt = x.T
    xdiv = (x >> 2).T
    xct = x_char.transpose(1, 2, 0)
    wt4 = word_table.reshape(WTROWS, 4 * D)
    cb = char_table.T.astype(jnp.bfloat16)
    lo = lax.bitcast_convert_type(cb[:LANES], jnp.uint16).astype(jnp.uint32)
    hi = lax.bitcast_convert_type(cb[LANES:], jnp.uint16).astype(jnp.uint32)
    packed = lax.bitcast_convert_type(lo | (hi << 16), jnp.int32)
    ctp = jnp.pad(packed, ((0, 0), (0, CT_PITCH - CHAR_VOC))).reshape(-1)
    out_t = _emb_kernel(xt, xdiv, xct, wt4, ctp)
    return out_t.T
